# baseline (speedup 1.0000x reference)
import jax
import jax.numpy as jnp
from jax.experimental import pallas as pl


def _gcn_kernel(x_ref, adj_ref, w1_ref, b1_ref, w2_ref, b2_ref, out_ref):
    a = adj_ref[...]
    n = a.shape[0]
    ones_row = jnp.ones((1, n), jnp.float32)
    deg = jnp.dot(ones_row, a, preferred_element_type=jnp.float32) + 1.0
    dinv = jax.lax.rsqrt(deg)

    gT = jax.lax.dot_general(
        w1_ref[...], x_ref[...], (((0,), (1,)), ((), ())),
        preferred_element_type=jnp.float32,
    )
    uT = gT * dinv

    tT = jnp.dot(uT, a, preferred_element_type=jnp.float32) + uT
    h1T = jnp.maximum(tT * dinv + b1_ref[...].T, 0.0)

    vT = jax.lax.dot_general(
        w2_ref[...], h1T, (((0,), (0,)), ((), ())),
        preferred_element_type=jnp.float32,
    ) * dinv

    sT = jnp.dot(vT, a, preferred_element_type=jnp.float32) + vT
    oT = sT * dinv + b2_ref[...].T

    m = jnp.max(oT, axis=0, keepdims=True)
    e = jnp.exp(oT - m)
    lse = jnp.log(jnp.sum(e, axis=0, keepdims=True)) + m
    out_ref[...] = (oT - lse).T


def kernel(x, adj, W1, b1, W2, b2):
    n = x.shape[0]
    nclass = W2.shape[1]
    return pl.pallas_call(
        _gcn_kernel,
        out_shape=jax.ShapeDtypeStruct((n, nclass), jnp.float32),
    )(x, adj, W1, b1.reshape(1, -1), W2, b2.reshape(1, -1))
